# baseline (speedup 1.0000x reference)
import functools

import jax
import jax.numpy as jnp
from jax import lax
from jax.experimental import pallas as pl
from jax.experimental.pallas import tpu as pltpu
from jax.experimental.pallas import tpu_sc as plsc

_FIELD_OFFSET = 1000000
_B = 4096
_D = 16
_ROWS = 2000000

_NC = 2
_NS = 16
_L = 16
_NW = _NC * _NS
_BPW = _B // _NW

_TILE_R = 128
_TILE_D = 8
_RT = _ROWS // _TILE_R
_DCONST = [(d // _TILE_D) * _RT * 1024 + (d % _TILE_D) * _TILE_R
           for d in range(_D)]


def _mf_body(x_hbm, t_hbm, out_hbm, xv, ub, ib, idxb, gath, outv, sem):
    wid = lax.axis_index("s") * _NC + lax.axis_index("c")

    pltpu.sync_copy(x_hbm.at[pl.ds(wid * 2 * _BPW, 2 * _BPW)], xv)

    for blk in range(_BPW // _L):
        sl = pl.ds(blk * _L, _L)
        u = xv[sl]
        it = xv[pl.ds(_BPW + blk * _L, _L)] + _FIELD_OFFSET
        ub[sl] = ((u >> 7) << 10) + (u & 127)
        ib[sl] = ((it >> 7) << 10) + (it & 127)

    for blk in range(_BPW // _L):
        sl = pl.ds(blk * _L, _L)
        uv = ub[sl]
        iv = ib[sl]
        for d in range(_D):
            idxb[d, sl] = uv + _DCONST[d]
            idxb[_D + d, sl] = iv + _DCONST[d]

    copies = [
        pltpu.async_copy(t_hbm.at[idxb.at[k]], gath.at[k], sem)
        for k in range(2 * _D)
    ]
    for c in copies:
        c.wait()

    for blk in range(_BPW // _L):
        sl = pl.ds(blk * _L, _L)
        acc = gath[0, sl] * gath[_D, sl]
        for d in range(1, _D):
            acc = acc + gath[d, sl] * gath[_D + d, sl]
        outv[sl] = acc

    pltpu.sync_copy(outv, out_hbm.at[pl.ds(wid * _BPW, _BPW)])


@functools.partial(
    pl.kernel,
    out_type=jax.ShapeDtypeStruct((_B,), jnp.float32),
    mesh=plsc.VectorSubcoreMesh(core_axis_name="c", subcore_axis_name="s"),
    compiler_params=pltpu.CompilerParams(
        needs_layout_passes=False, use_tc_tiling_on_sc=False
    ),
    scratch_types=[
        pltpu.VMEM((2 * _BPW,), jnp.int32),
        pltpu.VMEM((_BPW,), jnp.int32),
        pltpu.VMEM((_BPW,), jnp.int32),
        pltpu.VMEM((2 * _D, _BPW), jnp.int32),
        pltpu.VMEM((2 * _D, _BPW), jnp.float32),
        pltpu.VMEM((_BPW,), jnp.float32),
        pltpu.SemaphoreType.DMA,
    ],
)
def _mf_kernel(x_hbm, t_hbm, out_hbm, xv, ub, ib, idxb, gath, outv, sem):
    _mf_body(x_hbm, t_hbm, out_hbm, xv, ub, ib, idxb, gath, outv, sem)


def kernel(x, table):
    xflat = (
        x.reshape(_B // _TILE_R, _TILE_R, 2)
        .transpose(0, 2, 1)
        .reshape(2 * _B)
    )
    tflat = (
        table.reshape(_RT, _TILE_R, _D // _TILE_D, _TILE_D)
        .transpose(2, 0, 3, 1)
        .reshape(_ROWS * _D)
    )
    y = _mf_kernel(xflat, tflat)
    return y.reshape(_B, 1)
